# baseline (speedup 1.0000x reference)
import jax
import jax.numpy as jnp
from jax import lax
from jax.experimental import pallas as pl
from jax.experimental.pallas import tpu as pltpu
from jax.experimental.pallas import tpu_sc as plsc

B = 16384
D = 64
NUM_CORES = 2
NUM_SUBCORES = 16
NUM_WORKERS = NUM_CORES * NUM_SUBCORES
BPW = B // NUM_WORKERS
IDX_CHUNK = 128
NCHUNK = BPW // IDX_CHUNK
LANES = 16
NVEC = BPW // LANES


def _bpr_body(users_hbm, item_i_hbm, item_j_hbm, user_emb_hbm, item_emb_hbm,
              pos_hbm, neg_hbm,
              idx_u, idx_i, idx_j, u_rows, i_rows, j_rows, pos_v, neg_v, sem):
    cid = lax.axis_index("c")
    sid = lax.axis_index("s")
    wid = sid * NUM_CORES + cid

    pltpu.sync_copy(users_hbm.at[wid], idx_u)
    pltpu.sync_copy(item_i_hbm.at[wid], idx_i)
    pltpu.sync_copy(item_j_hbm.at[wid], idx_j)

    copies = []
    for k in range(NCHUNK):
        rows = pl.ds(k * IDX_CHUNK, IDX_CHUNK)
        copies.append(pltpu.async_copy(
            user_emb_hbm.at[idx_u.at[k]], u_rows.at[rows], sem))
        copies.append(pltpu.async_copy(
            item_emb_hbm.at[idx_i.at[k]], i_rows.at[rows], sem))
        copies.append(pltpu.async_copy(
            item_emb_hbm.at[idx_j.at[k]], j_rows.at[rows], sem))
    for c in copies:
        c.wait()

    lane_iota = lax.iota(jnp.int32, LANES)

    def chunk_body(c, _):
        base = c * LANES
        outp = jnp.zeros((LANES,), jnp.float32)
        outn = jnp.zeros((LANES,), jnp.float32)
        for e in range(LANES):
            b = base + e
            accp = jnp.zeros((LANES,), jnp.float32)
            accn = jnp.zeros((LANES,), jnp.float32)
            for d in range(0, D, LANES):
                sl = pl.ds(d, LANES)
                u = u_rows[b, sl]
                accp = accp + u * i_rows[b, sl]
                accn = accn + u * j_rows[b, sl]
            lane = lane_iota == e
            outp = jnp.where(lane, jnp.sum(accp), outp)
            outn = jnp.where(lane, jnp.sum(accn), outn)
        pos_v[pl.ds(base, LANES)] = outp
        neg_v[pl.ds(base, LANES)] = outn
        return ()

    lax.fori_loop(0, NVEC, chunk_body, ())

    out = pl.ds(wid * BPW, BPW)
    pltpu.sync_copy(pos_v, pos_hbm.at[out])
    pltpu.sync_copy(neg_v, neg_hbm.at[out])


@jax.jit
def _bpr(users3, item_i3, item_j3, user_emb, item_emb):
    mesh = plsc.VectorSubcoreMesh(core_axis_name="c", subcore_axis_name="s")
    f = pl.kernel(
        _bpr_body,
        out_type=(
            jax.ShapeDtypeStruct((B,), jnp.float32),
            jax.ShapeDtypeStruct((B,), jnp.float32),
        ),
        mesh=mesh,
        compiler_params=pltpu.CompilerParams(
            needs_layout_passes=False, use_tc_tiling_on_sc=False),
        scratch_types=[
            pltpu.VMEM((NCHUNK, IDX_CHUNK), jnp.int32),
            pltpu.VMEM((NCHUNK, IDX_CHUNK), jnp.int32),
            pltpu.VMEM((NCHUNK, IDX_CHUNK), jnp.int32),
            pltpu.VMEM((BPW, D), jnp.float32),
            pltpu.VMEM((BPW, D), jnp.float32),
            pltpu.VMEM((BPW, D), jnp.float32),
            pltpu.VMEM((BPW,), jnp.float32),
            pltpu.VMEM((BPW,), jnp.float32),
            pltpu.SemaphoreType.DMA,
        ],
    )
    return f(users3, item_i3, item_j3, user_emb, item_emb)


def kernel(users, item_i, item_j, user_emb, item_emb):
    users3 = users.astype(jnp.int32).reshape(NUM_WORKERS, NCHUNK, IDX_CHUNK)
    item_i3 = item_i.astype(jnp.int32).reshape(NUM_WORKERS, NCHUNK, IDX_CHUNK)
    item_j3 = item_j.astype(jnp.int32).reshape(NUM_WORKERS, NCHUNK, IDX_CHUNK)
    return _bpr(users3, item_i3, item_j3, user_emb, item_emb)
